# Initial kernel scaffold; baseline (speedup 1.0000x reference)
import jax
import jax.numpy as jnp
from jax.experimental import pallas as pl


def kernel(node_emb, w1, root1, b1, w2, root2, b2, edge_index, edge_type):
    raise NotImplementedError("write your pallas kernel here")


# speedup vs baseline: 3.0949x; 3.0949x over previous
import functools

import jax
import jax.numpy as jnp
from jax import lax
from jax.experimental import pallas as pl
from jax.experimental.pallas import tpu as pltpu
from jax.experimental.pallas import tpu_sc as plsc

N = 50000
H = 80
DP = 128
R = 8
NB = 5
BS = 16
E = 800000

TPS = 16
QN = N // 4
QROWS = 12544
STRIPE = QROWS // TPS
QCHUNK = STRIPE // 14
EK = 80

_mesh = plsc.VectorSubcoreMesh(core_axis_name="c", subcore_axis_name="s")


def _sc_counts(dst_hbm, typ_hbm, out_hbm, dv, tv, kv, ones_v, stage, cnt_sh, sem):
    cid = lax.axis_index("c")
    tid = lax.axis_index("s")
    wid = cid * TPS + tid
    per_tile_words = (R * N) // TPS

    z = jnp.zeros((16,), jnp.float32)

    def zbody(i, _):
        stage[pl.ds(i * 16, 16)] = z
        return 0

    lax.fori_loop(0, 64, zbody, 0)
    nfull = per_tile_words // 1024
    for q in range(nfull):
        pltpu.sync_copy(stage.at[pl.ds(0, 1024)],
                        cnt_sh.at[pl.ds(tid * per_tile_words + q * 1024, 1024)])
    tail = per_tile_words - nfull * 1024
    pltpu.sync_copy(stage.at[pl.ds(0, tail)],
                    cnt_sh.at[pl.ds(tid * per_tile_words + nfull * 1024, tail)])

    for i in range(8):
        ones_v[pl.ds(i * 16, 16)] = jnp.ones((16,), jnp.float32)

    plsc.subcore_barrier()

    nchunks = jnp.where(wid < (E // 128) % 32, (E // 128) // 32 + 1,
                        (E // 128) // 32)

    def body(j, _):
        off = (j * 32 + wid) * 128
        pltpu.sync_copy(dst_hbm.at[pl.ds(off, 128)], dv)
        pltpu.sync_copy(typ_hbm.at[pl.ds(off, 128)], tv)
        for g in range(8):
            sl = pl.ds(g * 16, 16)
            kv[sl] = tv[sl] * N + dv[sl]
        pltpu.sync_copy(ones_v, cnt_sh.at[kv], add=True)
        return 0

    lax.fori_loop(0, nchunks, body, 0)

    plsc.subcore_barrier()

    plane = cid * (R * N)
    for q in range(nfull):
        o = tid * per_tile_words + q * 1024
        pltpu.sync_copy(cnt_sh.at[pl.ds(o, 1024)], stage.at[pl.ds(0, 1024)])
        pltpu.sync_copy(stage.at[pl.ds(0, 1024)], out_hbm.at[pl.ds(plane + o, 1024)])
    o = tid * per_tile_words + nfull * 1024
    pltpu.sync_copy(cnt_sh.at[pl.ds(o, tail)], stage.at[pl.ds(0, tail)])
    pltpu.sync_copy(stage.at[pl.ds(0, tail)], out_hbm.at[pl.ds(plane + o, tail)])


_counts_call = pl.kernel(
    _sc_counts,
    out_type=jax.ShapeDtypeStruct((2 * R * N,), jnp.float32),
    mesh=_mesh,
    scratch_types=[
        pltpu.VMEM((128,), jnp.int32),
        pltpu.VMEM((128,), jnp.int32),
        pltpu.VMEM((128,), jnp.int32),
        pltpu.VMEM((128,), jnp.float32),
        pltpu.VMEM((1024,), jnp.float32),
        pltpu.VMEM_SHARED((R * N,), jnp.float32),
        pltpu.SemaphoreType.DMA,
    ],
)


def _sc_scale(cnt_hbm, dst_hbm, typ_hbm, s_hbm, dv, tv, kv, k2v, c0, c1, sv, sem):
    cid = lax.axis_index("c")
    tid = lax.axis_index("s")
    wid = cid * TPS + tid
    nchunks = jnp.where(wid < (E // 128) % 32, (E // 128) // 32 + 1,
                        (E // 128) // 32)

    def body(j, _):
        off = (j * 32 + wid) * 128
        pltpu.sync_copy(dst_hbm.at[pl.ds(off, 128)], dv)
        pltpu.sync_copy(typ_hbm.at[pl.ds(off, 128)], tv)
        for g in range(8):
            sl = pl.ds(g * 16, 16)
            key = tv[sl] * N + dv[sl]
            kv[sl] = key
            k2v[sl] = key + R * N
        pltpu.async_copy(cnt_hbm.at[kv], c0, sem).wait()
        pltpu.async_copy(cnt_hbm.at[k2v], c1, sem).wait()
        for g in range(8):
            sl = pl.ds(g * 16, 16)
            tot = c0[sl] + c1[sl]
            sv[sl] = 1.0 / jnp.maximum(tot, 1.0)
        pltpu.sync_copy(sv, s_hbm.at[pl.ds(off, 128)])
        return 0

    lax.fori_loop(0, nchunks, body, 0)


_scale_call = pl.kernel(
    _sc_scale,
    out_type=jax.ShapeDtypeStruct((E,), jnp.float32),
    mesh=_mesh,
    scratch_types=[
        pltpu.VMEM((128,), jnp.int32),
        pltpu.VMEM((128,), jnp.int32),
        pltpu.VMEM((128,), jnp.int32),
        pltpu.VMEM((128,), jnp.int32),
        pltpu.VMEM((128,), jnp.float32),
        pltpu.VMEM((128,), jnp.float32),
        pltpu.VMEM((128,), jnp.float32),
        pltpu.SemaphoreType.DMA,
    ],
)


def _sc_msgpass(y_hbm, src_hbm, dst_hbm, typ_hbm, s_hbm, rpad_hbm, out_hbm,
                srcv, dv, tv, kv, dlv, sv, rows, stage, acc, sem):
    cid = lax.axis_index("c")
    tid = lax.axis_index("s")
    lane = lax.iota(jnp.int32, 16)

    edges_per_tile = E // TPS
    ebase = tid * edges_per_tile
    dump_row = QN + tid

    for q in range(2):
        qi = cid * 2 + q
        qbase = qi * QN

        for p in range(14):
            r0 = tid * STRIPE + p * QCHUNK
            pltpu.sync_copy(rpad_hbm.at[qi, pl.ds(r0, QCHUNK), :], stage)
            pltpu.sync_copy(stage, acc.at[pl.ds(r0, QCHUNK), :])

        plsc.subcore_barrier()

        def body(j, _):
            off = ebase + j * EK
            pltpu.sync_copy(src_hbm.at[pl.ds(off, EK)], srcv)
            pltpu.sync_copy(dst_hbm.at[pl.ds(off, EK)], dv)
            pltpu.sync_copy(typ_hbm.at[pl.ds(off, EK)], tv)
            pltpu.sync_copy(s_hbm.at[pl.ds(off, EK)], sv)
            for g in range(EK // 16):
                sl = pl.ds(g * 16, 16)
                kv[sl] = tv[sl] * N + srcv[sl]
                local = dv[sl] - qbase
                own = (local >= 0) & (local < QN)
                dlv[sl] = jnp.where(own, local, dump_row)
            pltpu.async_copy(y_hbm.at[kv], rows, sem).wait()
            for g in range(EK // 16):
                svec = sv[pl.ds(g * 16, 16)]
                for l in range(16):
                    e = g * 16 + l
                    scal = lax.gather(
                        svec, (lane * 0 + l).reshape(16, 1),
                        lax.GatherDimensionNumbers(
                            offset_dims=(), collapsed_slice_dims=(0,),
                            start_index_map=(0,)),
                        (1,), mode=lax.GatherScatterMode.PROMISE_IN_BOUNDS)
                    for b in range(NB):
                        sl = pl.ds(b * 16, 16)
                        rows[e, sl] = rows[e, sl] * scal
            pltpu.sync_copy(rows, acc.at[dlv], add=True)
            return 0

        lax.fori_loop(0, edges_per_tile // EK, body, 0)

        plsc.subcore_barrier()

        for p in range(14):
            r0 = tid * STRIPE + p * QCHUNK
            pltpu.sync_copy(acc.at[pl.ds(r0, QCHUNK), :], stage)
            pltpu.sync_copy(stage, out_hbm.at[qi, pl.ds(r0, QCHUNK), :])

        if q == 0:
            plsc.subcore_barrier()


_msgpass_call = pl.kernel(
    _sc_msgpass,
    out_type=jax.ShapeDtypeStruct((4, QROWS, DP), jnp.float32),
    mesh=_mesh,
    scratch_types=[
        pltpu.VMEM((EK,), jnp.int32),
        pltpu.VMEM((EK,), jnp.int32),
        pltpu.VMEM((EK,), jnp.int32),
        pltpu.VMEM((EK,), jnp.int32),
        pltpu.VMEM((EK,), jnp.int32),
        pltpu.VMEM((EK,), jnp.float32),
        pltpu.VMEM((EK, DP), jnp.float32),
        pltpu.VMEM((QCHUNK, DP), jnp.float32),
        pltpu.VMEM_SHARED((QROWS, DP), jnp.float32),
        pltpu.SemaphoreType.DMA,
    ],
)


def _tc_transform(x_ref, w_ref, root_ref, b_ref, y_ref, r_ref, *, relu_in):
    r = pl.program_id(1)
    x = x_ref[...]
    if relu_in:
        x = jnp.maximum(x, 0.0)
    parts = []
    for b in range(NB):
        xb = x[:, b * BS:(b + 1) * BS]
        parts.append(
            lax.dot_general(xb, w_ref[r, b],
                            (((1,), (0,)), ((), ())),
                            precision=lax.Precision.HIGHEST))
    parts.append(jnp.zeros((x.shape[0], DP - H), jnp.float32))
    y_ref[...] = jnp.concatenate(parts, axis=1)

    @pl.when(r == 0)
    def _():
        r_ref[...] = (
            lax.dot_general(x, root_ref[...], (((1,), (0,)), ((), ())),
                            precision=lax.Precision.HIGHEST)
            + b_ref[...]
        )


def _transform(x, w, root, b, relu_in):
    NT = 200
    call = pl.pallas_call(
        functools.partial(_tc_transform, relu_in=relu_in),
        grid=(N // NT, R),
        in_specs=[
            pl.BlockSpec((NT, H), lambda i, r: (i, 0)),
            pl.BlockSpec((R, NB, BS, BS), lambda i, r: (0, 0, 0, 0)),
            pl.BlockSpec((H, H), lambda i, r: (0, 0)),
            pl.BlockSpec((1, H), lambda i, r: (0, 0)),
        ],
        out_specs=[
            pl.BlockSpec((NT, DP), lambda i, r: (r * (N // NT) + i, 0)),
            pl.BlockSpec((NT, H), lambda i, r: (i, 0)),
        ],
        out_shape=[
            jax.ShapeDtypeStruct((R * N, DP), jnp.float32),
            jax.ShapeDtypeStruct((N, H), jnp.float32),
        ],
    )
    return call(x, w, root, b.reshape(1, H))


def _pad_root(rterm):
    quarters = rterm.reshape(4, QN, H)
    return jnp.pad(quarters, ((0, 0), (0, QROWS - QN), (0, DP - H)))


def kernel(node_emb, w1, root1, b1, w2, root2, b2, edge_index, edge_type):
    src = edge_index[0].astype(jnp.int32)
    dst = edge_index[1].astype(jnp.int32)
    typ = edge_type.astype(jnp.int32)

    cnt = _counts_call(dst, typ)
    s = _scale_call(cnt, dst, typ)

    y1, rt1 = _transform(node_emb, w1, root1, b1, relu_in=False)
    conv1 = _msgpass_call(y1, src, dst, typ, s, _pad_root(rt1))
    x1 = conv1[:, :QN, :H].reshape(N, H)

    y2, rt2 = _transform(x1, w2, root2, b2, relu_in=True)
    conv2 = _msgpass_call(y2, src, dst, typ, s, _pad_root(rt2))
    return conv2[:, :QN, :H].reshape(N, H)


# speedup vs baseline: 4.4768x; 1.4465x over previous
import functools

import jax
import jax.numpy as jnp
from jax import lax
from jax.experimental import pallas as pl
from jax.experimental.pallas import tpu as pltpu
from jax.experimental.pallas import tpu_sc as plsc

N = 50000
H = 80
DP = 128
R = 8
NB = 5
BS = 16
E = 800000

TPS = 16
QN = N // 4
QROWS = 12544
STRIPE = QROWS // TPS
QCHUNK = STRIPE // 14
NINIT = 14
EK = 80

_mesh = plsc.VectorSubcoreMesh(core_axis_name="c", subcore_axis_name="s")


def _sc_counts(dst_hbm, typ_hbm, out_hbm, dv, tv, kv, ones_v, stage, cnt_sh, sem):
    cid = lax.axis_index("c")
    tid = lax.axis_index("s")
    wid = cid * TPS + tid
    per_tile_words = (R * N) // TPS

    z = jnp.zeros((16,), jnp.float32)

    def zbody(i, _):
        stage[pl.ds(i * 16, 16)] = z
        return 0

    lax.fori_loop(0, 64, zbody, 0)
    nfull = per_tile_words // 1024
    for q in range(nfull):
        pltpu.sync_copy(stage.at[pl.ds(0, 1024)],
                        cnt_sh.at[pl.ds(tid * per_tile_words + q * 1024, 1024)])
    tail = per_tile_words - nfull * 1024
    pltpu.sync_copy(stage.at[pl.ds(0, tail)],
                    cnt_sh.at[pl.ds(tid * per_tile_words + nfull * 1024, tail)])

    for i in range(8):
        ones_v[pl.ds(i * 16, 16)] = jnp.ones((16,), jnp.float32)

    plsc.subcore_barrier()

    nchunks = jnp.where(wid < (E // 128) % 32, (E // 128) // 32 + 1,
                        (E // 128) // 32)

    def body(j, _):
        off = (j * 32 + wid) * 128
        pltpu.sync_copy(dst_hbm.at[pl.ds(off, 128)], dv)
        pltpu.sync_copy(typ_hbm.at[pl.ds(off, 128)], tv)
        for g in range(8):
            sl = pl.ds(g * 16, 16)
            kv[sl] = tv[sl] * N + dv[sl]
        pltpu.sync_copy(ones_v, cnt_sh.at[kv], add=True)
        return 0

    lax.fori_loop(0, nchunks, body, 0)

    plsc.subcore_barrier()

    plane = cid * (R * N)
    for q in range(nfull):
        o = tid * per_tile_words + q * 1024
        pltpu.sync_copy(cnt_sh.at[pl.ds(o, 1024)], stage.at[pl.ds(0, 1024)])
        pltpu.sync_copy(stage.at[pl.ds(0, 1024)], out_hbm.at[pl.ds(plane + o, 1024)])
    o = tid * per_tile_words + nfull * 1024
    pltpu.sync_copy(cnt_sh.at[pl.ds(o, tail)], stage.at[pl.ds(0, tail)])
    pltpu.sync_copy(stage.at[pl.ds(0, tail)], out_hbm.at[pl.ds(plane + o, tail)])


_counts_call = pl.kernel(
    _sc_counts,
    out_type=jax.ShapeDtypeStruct((2 * R * N,), jnp.float32),
    mesh=_mesh,
    scratch_types=[
        pltpu.VMEM((128,), jnp.int32),
        pltpu.VMEM((128,), jnp.int32),
        pltpu.VMEM((128,), jnp.int32),
        pltpu.VMEM((128,), jnp.float32),
        pltpu.VMEM((1024,), jnp.float32),
        pltpu.VMEM_SHARED((R * N,), jnp.float32),
        pltpu.SemaphoreType.DMA,
    ],
)


def _sc_scale(cnt_hbm, dst_hbm, typ_hbm, s_hbm, dv, tv, kv, k2v, c0, c1, sv, sem):
    cid = lax.axis_index("c")
    tid = lax.axis_index("s")
    wid = cid * TPS + tid
    nchunks = jnp.where(wid < (E // 128) % 32, (E // 128) // 32 + 1,
                        (E // 128) // 32)

    def body(j, _):
        off = (j * 32 + wid) * 128
        pltpu.sync_copy(dst_hbm.at[pl.ds(off, 128)], dv)
        pltpu.sync_copy(typ_hbm.at[pl.ds(off, 128)], tv)
        for g in range(8):
            sl = pl.ds(g * 16, 16)
            key = tv[sl] * N + dv[sl]
            kv[sl] = key
            k2v[sl] = key + R * N
        pltpu.async_copy(cnt_hbm.at[kv], c0, sem).wait()
        pltpu.async_copy(cnt_hbm.at[k2v], c1, sem).wait()
        for g in range(8):
            sl = pl.ds(g * 16, 16)
            tot = c0[sl] + c1[sl]
            sv[sl] = 1.0 / jnp.maximum(tot, 1.0)
        pltpu.sync_copy(sv, s_hbm.at[pl.ds(off, 128)])
        return 0

    lax.fori_loop(0, nchunks, body, 0)


_scale_call = pl.kernel(
    _sc_scale,
    out_type=jax.ShapeDtypeStruct((E,), jnp.float32),
    mesh=_mesh,
    scratch_types=[
        pltpu.VMEM((128,), jnp.int32),
        pltpu.VMEM((128,), jnp.int32),
        pltpu.VMEM((128,), jnp.int32),
        pltpu.VMEM((128,), jnp.int32),
        pltpu.VMEM((128,), jnp.float32),
        pltpu.VMEM((128,), jnp.float32),
        pltpu.VMEM((128,), jnp.float32),
        pltpu.SemaphoreType.DMA,
    ],
)


def _sc_msgpass(y_hbm, src_hbm, dst_hbm, typ_hbm, s_hbm, rpad_hbm, out_hbm,
                srcv0, dv0, tv0, sv0, kv0, dlv0, rows0,
                srcv1, dv1, tv1, sv1, kv1, dlv1, rows1,
                stage, acc, semld0, semld1, semg0, semg1):
    cid = lax.axis_index("c")
    tid = lax.axis_index("s")
    lane = lax.iota(jnp.int32, 16)

    edges_per_tile = E // TPS
    NCH = edges_per_tile // EK
    ebase = tid * edges_per_tile
    dump_row = QN + tid

    bufs = ((srcv0, dv0, tv0, sv0, kv0, dlv0, rows0, semld0, semg0),
            (srcv1, dv1, tv1, sv1, kv1, dlv1, rows1, semld1, semg1))

    def fire_loads(b, off):
        srcv, dv, tv, sv, _, _, _, semld, _ = bufs[b]
        pltpu.async_copy(src_hbm.at[pl.ds(off, EK)], srcv, semld)
        pltpu.async_copy(dst_hbm.at[pl.ds(off, EK)], dv, semld)
        pltpu.async_copy(typ_hbm.at[pl.ds(off, EK)], tv, semld)
        pltpu.async_copy(s_hbm.at[pl.ds(off, EK)], sv, semld)

    def wait_loads(b, off):
        srcv, dv, tv, sv, _, _, _, semld, _ = bufs[b]
        pltpu.make_async_copy(src_hbm.at[pl.ds(off, EK)], srcv, semld).wait()
        pltpu.make_async_copy(dst_hbm.at[pl.ds(off, EK)], dv, semld).wait()
        pltpu.make_async_copy(typ_hbm.at[pl.ds(off, EK)], tv, semld).wait()
        pltpu.make_async_copy(s_hbm.at[pl.ds(off, EK)], sv, semld).wait()

    def finish(b, qbase):
        srcv, dv, tv, sv, kv, dlv, rows, _, semg = bufs[b]
        for g in range(EK // 16):
            sl = pl.ds(g * 16, 16)
            kv[sl] = tv[sl] * N + srcv[sl]
            local = dv[sl] - qbase
            own = (local >= 0) & (local < QN)
            dlv[sl] = jnp.where(own, local, dump_row)
        return pltpu.async_copy(y_hbm.at[kv], rows, semg)

    def drain(b):
        srcv, dv, tv, sv, kv, dlv, rows, _, semg = bufs[b]
        pltpu.make_async_copy(y_hbm.at[kv], rows, semg).wait()
        for g in range(EK // 16):
            svec = sv[pl.ds(g * 16, 16)]
            for l in range(16):
                e = g * 16 + l
                scal = lax.gather(
                    svec, (lane * 0 + l).reshape(16, 1),
                    lax.GatherDimensionNumbers(
                        offset_dims=(), collapsed_slice_dims=(0,),
                        start_index_map=(0,)),
                    (1,), mode=lax.GatherScatterMode.PROMISE_IN_BOUNDS)
                for nb in range(NB):
                    sl = pl.ds(nb * 16, 16)
                    rows[e, sl] = rows[e, sl] * scal
        pltpu.sync_copy(rows, acc.at[dlv], add=True)

    for q in range(2):
        qi = cid * 2 + q
        qbase = qi * QN

        for p in range(NINIT):
            r0 = tid * STRIPE + p * QCHUNK
            pltpu.sync_copy(rpad_hbm.at[qi, pl.ds(r0, QCHUNK), :], stage)
            pltpu.sync_copy(stage, acc.at[pl.ds(r0, QCHUNK), :])

        plsc.subcore_barrier()

        fire_loads(0, ebase)

        def body(j, _):
            for b in range(2):
                jj = 2 * j + b
                off = ebase + jj * EK
                wait_loads(b, off)
                finish(b, qbase)
                fire_loads(1 - b, off + EK)
                drain(b)
            return 0

        lax.fori_loop(0, (NCH - 1) // 2, body, 0)
        off_last = ebase + (NCH - 1) * EK
        wait_loads(0, off_last)
        finish(0, qbase)
        drain(0)

        plsc.subcore_barrier()

        for p in range(NINIT):
            r0 = tid * STRIPE + p * QCHUNK
            pltpu.sync_copy(acc.at[pl.ds(r0, QCHUNK), :], stage)
            pltpu.sync_copy(stage, out_hbm.at[qi, pl.ds(r0, QCHUNK), :])

        if q == 0:
            plsc.subcore_barrier()


def _edge_bufs():
    return [
        pltpu.VMEM((EK,), jnp.int32),
        pltpu.VMEM((EK,), jnp.int32),
        pltpu.VMEM((EK,), jnp.int32),
        pltpu.VMEM((EK,), jnp.float32),
        pltpu.VMEM((EK,), jnp.int32),
        pltpu.VMEM((EK,), jnp.int32),
        pltpu.VMEM((EK, DP), jnp.float32),
    ]


_msgpass_call = pl.kernel(
    _sc_msgpass,
    out_type=jax.ShapeDtypeStruct((4, QROWS, DP), jnp.float32),
    mesh=_mesh,
    scratch_types=(
        _edge_bufs() + _edge_bufs() + [
            pltpu.VMEM((QCHUNK, DP), jnp.float32),
            pltpu.VMEM_SHARED((QROWS, DP), jnp.float32),
            pltpu.SemaphoreType.DMA,
            pltpu.SemaphoreType.DMA,
            pltpu.SemaphoreType.DMA,
            pltpu.SemaphoreType.DMA,
        ]
    ),
)


def _tc_transform(x_ref, w_ref, root_ref, b_ref, y_ref, r_ref, *, relu_in):
    r = pl.program_id(1)
    x = x_ref[...]
    if relu_in:
        x = jnp.maximum(x, 0.0)
    parts = []
    for b in range(NB):
        xb = x[:, b * BS:(b + 1) * BS]
        parts.append(
            lax.dot_general(xb, w_ref[r, b],
                            (((1,), (0,)), ((), ())),
                            precision=lax.Precision.HIGHEST))
    parts.append(jnp.zeros((x.shape[0], DP - H), jnp.float32))
    y_ref[...] = jnp.concatenate(parts, axis=1)

    @pl.when(r == 0)
    def _():
        r_ref[...] = (
            lax.dot_general(x, root_ref[...], (((1,), (0,)), ((), ())),
                            precision=lax.Precision.HIGHEST)
            + b_ref[...]
        )


def _transform(x, w, root, b, relu_in):
    NT = 200
    call = pl.pallas_call(
        functools.partial(_tc_transform, relu_in=relu_in),
        grid=(N // NT, R),
        in_specs=[
            pl.BlockSpec((NT, H), lambda i, r: (i, 0)),
            pl.BlockSpec((R, NB, BS, BS), lambda i, r: (0, 0, 0, 0)),
            pl.BlockSpec((H, H), lambda i, r: (0, 0)),
            pl.BlockSpec((1, H), lambda i, r: (0, 0)),
        ],
        out_specs=[
            pl.BlockSpec((NT, DP), lambda i, r: (r * (N // NT) + i, 0)),
            pl.BlockSpec((NT, H), lambda i, r: (i, 0)),
        ],
        out_shape=[
            jax.ShapeDtypeStruct((R * N, DP), jnp.float32),
            jax.ShapeDtypeStruct((N, H), jnp.float32),
        ],
    )
    return call(x, w, root, b.reshape(1, H))


def _pad_root(rterm):
    quarters = rterm.reshape(4, QN, H)
    return jnp.pad(quarters, ((0, 0), (0, QROWS - QN), (0, DP - H)))


def kernel(node_emb, w1, root1, b1, w2, root2, b2, edge_index, edge_type):
    src = edge_index[0].astype(jnp.int32)
    dst = edge_index[1].astype(jnp.int32)
    typ = edge_type.astype(jnp.int32)

    cnt = _counts_call(dst, typ)
    s = _scale_call(cnt, dst, typ)

    y1, rt1 = _transform(node_emb, w1, root1, b1, relu_in=False)
    conv1 = _msgpass_call(y1, src, dst, typ, s, _pad_root(rt1))
    x1 = conv1[:, :QN, :H].reshape(N, H)

    y2, rt2 = _transform(x1, w2, root2, b2, relu_in=True)
    conv2 = _msgpass_call(y2, src, dst, typ, s, _pad_root(rt2))
    return conv2[:, :QN, :H].reshape(N, H)


# speedup vs baseline: 6.0356x; 1.3482x over previous
import functools

import jax
import jax.numpy as jnp
from jax import lax
from jax.experimental import pallas as pl
from jax.experimental.pallas import tpu as pltpu
from jax.experimental.pallas import tpu_sc as plsc

N = 50000
H = 80
DP = 128
R = 8
NB = 5
BS = 16
E = 800000

TPS = 16
QN = N // 4
QROWS = 12544
STRIPE = QROWS // TPS
QCHUNK = STRIPE // 14
NINIT = 14
EK = 80

_mesh = plsc.VectorSubcoreMesh(core_axis_name="c", subcore_axis_name="s")


def _sc_counts(dst_hbm, typ_hbm, out_hbm, dv, tv, kv, ones_v, stage, cnt_sh, sem):
    cid = lax.axis_index("c")
    tid = lax.axis_index("s")
    wid = cid * TPS + tid
    per_tile_words = (R * N) // TPS

    z = jnp.zeros((16,), jnp.float32)

    def zbody(i, _):
        stage[pl.ds(i * 16, 16)] = z
        return 0

    lax.fori_loop(0, 64, zbody, 0)
    nfull = per_tile_words // 1024
    for q in range(nfull):
        pltpu.sync_copy(stage.at[pl.ds(0, 1024)],
                        cnt_sh.at[pl.ds(tid * per_tile_words + q * 1024, 1024)])
    tail = per_tile_words - nfull * 1024
    pltpu.sync_copy(stage.at[pl.ds(0, tail)],
                    cnt_sh.at[pl.ds(tid * per_tile_words + nfull * 1024, tail)])

    for i in range(8):
        ones_v[pl.ds(i * 16, 16)] = jnp.ones((16,), jnp.float32)

    plsc.subcore_barrier()

    nchunks = jnp.where(wid < (E // 128) % 32, (E // 128) // 32 + 1,
                        (E // 128) // 32)

    def body(j, _):
        off = (j * 32 + wid) * 128
        pltpu.sync_copy(dst_hbm.at[pl.ds(off, 128)], dv)
        pltpu.sync_copy(typ_hbm.at[pl.ds(off, 128)], tv)
        for g in range(8):
            sl = pl.ds(g * 16, 16)
            kv[sl] = tv[sl] * N + dv[sl]
        pltpu.sync_copy(ones_v, cnt_sh.at[kv], add=True)
        return 0

    lax.fori_loop(0, nchunks, body, 0)

    plsc.subcore_barrier()

    plane = cid * (R * N)
    for q in range(nfull):
        o = tid * per_tile_words + q * 1024
        pltpu.sync_copy(cnt_sh.at[pl.ds(o, 1024)], stage.at[pl.ds(0, 1024)])
        pltpu.sync_copy(stage.at[pl.ds(0, 1024)], out_hbm.at[pl.ds(plane + o, 1024)])
    o = tid * per_tile_words + nfull * 1024
    pltpu.sync_copy(cnt_sh.at[pl.ds(o, tail)], stage.at[pl.ds(0, tail)])
    pltpu.sync_copy(stage.at[pl.ds(0, tail)], out_hbm.at[pl.ds(plane + o, tail)])


_counts_call = pl.kernel(
    _sc_counts,
    out_type=jax.ShapeDtypeStruct((2 * R * N,), jnp.float32),
    mesh=_mesh,
    scratch_types=[
        pltpu.VMEM((128,), jnp.int32),
        pltpu.VMEM((128,), jnp.int32),
        pltpu.VMEM((128,), jnp.int32),
        pltpu.VMEM((128,), jnp.float32),
        pltpu.VMEM((1024,), jnp.float32),
        pltpu.VMEM_SHARED((R * N,), jnp.float32),
        pltpu.SemaphoreType.DMA,
    ],
)


def _sc_scale(cnt_hbm, dst_hbm, typ_hbm, s_hbm, dv, tv, kv, k2v, c0, c1, sv, sem):
    cid = lax.axis_index("c")
    tid = lax.axis_index("s")
    wid = cid * TPS + tid
    nchunks = jnp.where(wid < (E // 128) % 32, (E // 128) // 32 + 1,
                        (E // 128) // 32)

    def body(j, _):
        off = (j * 32 + wid) * 128
        pltpu.sync_copy(dst_hbm.at[pl.ds(off, 128)], dv)
        pltpu.sync_copy(typ_hbm.at[pl.ds(off, 128)], tv)
        for g in range(8):
            sl = pl.ds(g * 16, 16)
            key = tv[sl] * N + dv[sl]
            kv[sl] = key
            k2v[sl] = key + R * N
        pltpu.async_copy(cnt_hbm.at[kv], c0, sem).wait()
        pltpu.async_copy(cnt_hbm.at[k2v], c1, sem).wait()
        for g in range(8):
            sl = pl.ds(g * 16, 16)
            tot = c0[sl] + c1[sl]
            sv[sl] = 1.0 / jnp.maximum(tot, 1.0)
        pltpu.sync_copy(sv, s_hbm.at[pl.ds(off, 128)])
        return 0

    lax.fori_loop(0, nchunks, body, 0)


_scale_call = pl.kernel(
    _sc_scale,
    out_type=jax.ShapeDtypeStruct((E,), jnp.float32),
    mesh=_mesh,
    scratch_types=[
        pltpu.VMEM((128,), jnp.int32),
        pltpu.VMEM((128,), jnp.int32),
        pltpu.VMEM((128,), jnp.int32),
        pltpu.VMEM((128,), jnp.int32),
        pltpu.VMEM((128,), jnp.float32),
        pltpu.VMEM((128,), jnp.float32),
        pltpu.VMEM((128,), jnp.float32),
        pltpu.SemaphoreType.DMA,
    ],
)


def _sc_msgpass(y_hbm, src_hbm, dst_hbm, typ_hbm, s_hbm, rpad_hbm, out_hbm,
                srcv0, dv0, tv0, sv0, kv0, dlv0, rows0,
                srcv1, dv1, tv1, sv1, kv1, dlv1, rows1,
                stage, acc, semld0, semld1, semg0, semg1):
    cid = lax.axis_index("c")
    tid = lax.axis_index("s")
    lane = lax.iota(jnp.int32, 16)

    edges_per_tile = E // TPS
    NCH = edges_per_tile // EK
    ebase = tid * edges_per_tile
    dump_row = QN + tid

    bufs = ((srcv0, dv0, tv0, sv0, kv0, dlv0, rows0, semld0, semg0),
            (srcv1, dv1, tv1, sv1, kv1, dlv1, rows1, semld1, semg1))

    def fire_loads(b, off):
        srcv, dv, tv, sv, _, _, _, semld, _ = bufs[b]
        pltpu.async_copy(src_hbm.at[pl.ds(off, EK)], srcv, semld)
        pltpu.async_copy(dst_hbm.at[pl.ds(off, EK)], dv, semld)
        pltpu.async_copy(typ_hbm.at[pl.ds(off, EK)], tv, semld)
        pltpu.async_copy(s_hbm.at[pl.ds(off, EK)], sv, semld)

    def wait_loads(b, off):
        srcv, dv, tv, sv, _, _, _, semld, _ = bufs[b]
        pltpu.make_async_copy(src_hbm.at[pl.ds(off, EK)], srcv, semld).wait()
        pltpu.make_async_copy(dst_hbm.at[pl.ds(off, EK)], dv, semld).wait()
        pltpu.make_async_copy(typ_hbm.at[pl.ds(off, EK)], tv, semld).wait()
        pltpu.make_async_copy(s_hbm.at[pl.ds(off, EK)], sv, semld).wait()

    def finish(b, qbase):
        srcv, dv, tv, sv, kv, dlv, rows, _, semg = bufs[b]
        for g in range(EK // 16):
            sl = pl.ds(g * 16, 16)
            kv[sl] = tv[sl] * N + srcv[sl]
            local = dv[sl] - qbase
            own = (local >= 0) & (local < QN)
            dlv[sl] = jnp.where(own, local, dump_row)
        return pltpu.async_copy(y_hbm.at[kv], rows, semg)

    def drain(b):
        srcv, dv, tv, sv, kv, dlv, rows, _, semg = bufs[b]
        pltpu.make_async_copy(y_hbm.at[kv], rows, semg).wait()
        for g in range(EK // 16):
            svec = sv[pl.ds(g * 16, 16)]
            for l in range(16):
                e = g * 16 + l
                scal = lax.gather(
                    svec, (lane * 0 + l).reshape(16, 1),
                    lax.GatherDimensionNumbers(
                        offset_dims=(), collapsed_slice_dims=(0,),
                        start_index_map=(0,)),
                    (1,), mode=lax.GatherScatterMode.PROMISE_IN_BOUNDS)
                for nb in range(NB):
                    sl = pl.ds(nb * 16, 16)
                    rows[e, sl] = rows[e, sl] * scal
        pltpu.sync_copy(rows, acc.at[dlv], add=True)

    for q in range(2):
        qi = cid * 2 + q
        qbase = qi * QN

        for p in range(NINIT):
            r0 = tid * STRIPE + p * QCHUNK
            pltpu.sync_copy(rpad_hbm.at[qi, pl.ds(r0, QCHUNK), :], stage)
            pltpu.sync_copy(stage, acc.at[pl.ds(r0, QCHUNK), :])

        plsc.subcore_barrier()

        fire_loads(0, ebase)

        def body(j, _):
            for b in range(2):
                jj = 2 * j + b
                off = ebase + jj * EK
                wait_loads(b, off)
                finish(b, qbase)
                fire_loads(1 - b, off + EK)
                drain(b)
            return 0

        lax.fori_loop(0, (NCH - 1) // 2, body, 0)
        off_last = ebase + (NCH - 1) * EK
        wait_loads(0, off_last)
        finish(0, qbase)
        drain(0)

        plsc.subcore_barrier()

        for p in range(NINIT):
            r0 = tid * STRIPE + p * QCHUNK
            pltpu.sync_copy(acc.at[pl.ds(r0, QCHUNK), :], stage)
            pltpu.sync_copy(stage, out_hbm.at[qi, pl.ds(r0, QCHUNK), :])

        if q == 0:
            plsc.subcore_barrier()


def _edge_bufs():
    return [
        pltpu.VMEM((EK,), jnp.int32),
        pltpu.VMEM((EK,), jnp.int32),
        pltpu.VMEM((EK,), jnp.int32),
        pltpu.VMEM((EK,), jnp.float32),
        pltpu.VMEM((EK,), jnp.int32),
        pltpu.VMEM((EK,), jnp.int32),
        pltpu.VMEM((EK, DP), jnp.float32),
    ]


_msgpass_call = pl.kernel(
    _sc_msgpass,
    out_type=jax.ShapeDtypeStruct((4, QROWS, DP), jnp.float32),
    mesh=_mesh,
    scratch_types=(
        _edge_bufs() + _edge_bufs() + [
            pltpu.VMEM((QCHUNK, DP), jnp.float32),
            pltpu.VMEM_SHARED((QROWS, DP), jnp.float32),
            pltpu.SemaphoreType.DMA,
            pltpu.SemaphoreType.DMA,
            pltpu.SemaphoreType.DMA,
            pltpu.SemaphoreType.DMA,
        ]
    ),
)


def _tc_transform(x_ref, w_ref, root_ref, b_ref, y_ref, r_ref, *, relu_in):
    r = pl.program_id(1)
    x = x_ref[...]
    if relu_in:
        x = jnp.maximum(x, 0.0)
    parts = []
    for b in range(NB):
        xb = x[:, b * BS:(b + 1) * BS]
        parts.append(
            lax.dot_general(xb, w_ref[r, b],
                            (((1,), (0,)), ((), ())),
                            precision=lax.Precision.HIGHEST))
    parts.append(jnp.zeros((x.shape[0], DP - H), jnp.float32))
    y_ref[...] = jnp.concatenate(parts, axis=1)

    @pl.when(r == 0)
    def _():
        r_ref[...] = (
            lax.dot_general(x, root_ref[...], (((1,), (0,)), ((), ())),
                            precision=lax.Precision.HIGHEST)
            + b_ref[...]
        )


def _transform(x, w, root, b, relu_in):
    NT = 2000
    call = pl.pallas_call(
        functools.partial(_tc_transform, relu_in=relu_in),
        grid=(N // NT, R),
        in_specs=[
            pl.BlockSpec((NT, H), lambda i, r: (i, 0)),
            pl.BlockSpec((R, NB, BS, BS), lambda i, r: (0, 0, 0, 0)),
            pl.BlockSpec((H, H), lambda i, r: (0, 0)),
            pl.BlockSpec((1, H), lambda i, r: (0, 0)),
        ],
        out_specs=[
            pl.BlockSpec((NT, DP), lambda i, r: (r * (N // NT) + i, 0)),
            pl.BlockSpec((NT, H), lambda i, r: (i, 0)),
        ],
        out_shape=[
            jax.ShapeDtypeStruct((R * N, DP), jnp.float32),
            jax.ShapeDtypeStruct((N, H), jnp.float32),
        ],
    )
    return call(x, w, root, b.reshape(1, H))


def _pad_root(rterm):
    quarters = rterm.reshape(4, QN, H)
    return jnp.pad(quarters, ((0, 0), (0, QROWS - QN), (0, DP - H)))


def kernel(node_emb, w1, root1, b1, w2, root2, b2, edge_index, edge_type):
    src = edge_index[0].astype(jnp.int32)
    dst = edge_index[1].astype(jnp.int32)
    typ = edge_type.astype(jnp.int32)

    cnt = _counts_call(dst, typ)
    s = _scale_call(cnt, dst, typ)

    y1, rt1 = _transform(node_emb, w1, root1, b1, relu_in=False)
    conv1 = _msgpass_call(y1, src, dst, typ, s, _pad_root(rt1))
    x1 = conv1[:, :QN, :H].reshape(N, H)

    y2, rt2 = _transform(x1, w2, root2, b2, relu_in=True)
    conv2 = _msgpass_call(y2, src, dst, typ, s, _pad_root(rt2))
    return conv2[:, :QN, :H].reshape(N, H)


# speedup vs baseline: 6.1289x; 1.0155x over previous
import functools

import jax
import jax.numpy as jnp
from jax import lax
from jax.experimental import pallas as pl
from jax.experimental.pallas import tpu as pltpu
from jax.experimental.pallas import tpu_sc as plsc

N = 50000
H = 80
DP = 128
R = 8
NB = 5
BS = 16
E = 800000

TPS = 16
QN = N // 4
QROWS = 12544
STRIPE = QROWS // TPS
QCHUNK = STRIPE // 14
NINIT = 14
EK = 80

_mesh = plsc.VectorSubcoreMesh(core_axis_name="c", subcore_axis_name="s")


def _sc_counts(dst_hbm, typ_hbm, out_hbm, dv, tv, kv, vals, dv1, tv1, kv1,
               vals1, stage, cnt_sh, semld, semld1):
    cid = lax.axis_index("c")
    tid = lax.axis_index("s")
    wid = cid * TPS + tid
    per_tile_words = (R * N) // TPS

    z = jnp.zeros((16,), jnp.float32)

    def zbody(i, _):
        stage[pl.ds(i * 16, 16)] = z
        return 0

    lax.fori_loop(0, 64, zbody, 0)
    nfull = per_tile_words // 1024
    for q in range(nfull):
        pltpu.sync_copy(stage.at[pl.ds(0, 1024)],
                        cnt_sh.at[pl.ds(tid * per_tile_words + q * 1024, 1024)])
    tail = per_tile_words - nfull * 1024
    pltpu.sync_copy(stage.at[pl.ds(0, tail)],
                    cnt_sh.at[pl.ds(tid * per_tile_words + nfull * 1024, tail)])

    plsc.subcore_barrier()

    bufs = ((dv, tv, kv, vals, semld), (dv1, tv1, kv1, vals1, semld1))

    def off_of(c):
        cidx = c * 32 + wid
        return jnp.where(cidx < E // 128, cidx * 128, 0), cidx < E // 128

    def fire(b, c):
        bdv, btv, _, _, sem = bufs[b]
        off, _ = off_of(c)
        pltpu.async_copy(dst_hbm.at[pl.ds(off, 128)], bdv, sem)
        pltpu.async_copy(typ_hbm.at[pl.ds(off, 128)], btv, sem)

    def wait(b, c):
        bdv, btv, _, _, sem = bufs[b]
        off, _ = off_of(c)
        pltpu.make_async_copy(dst_hbm.at[pl.ds(off, 128)], bdv, sem).wait()
        pltpu.make_async_copy(typ_hbm.at[pl.ds(off, 128)], btv, sem).wait()

    def process(b, c):
        bdv, btv, bkv, bvals, _ = bufs[b]
        _, valid = off_of(c)
        vf = jnp.where(valid, 1.0, 0.0)
        for g in range(8):
            sl = pl.ds(g * 16, 16)
            bkv[sl] = btv[sl] * N + bdv[sl]
            bvals[sl] = jnp.zeros((16,), jnp.float32) + vf
        pltpu.sync_copy(bvals, cnt_sh.at[bkv], add=True)

    fire(0, 0)

    def body(j, _):
        for b in range(2):
            c = 2 * j + b
            wait(b, c)
            fire(1 - b, c + 1)
            process(b, c)
        return 0

    lax.fori_loop(0, 98, body, 0)
    wait(0, 196)

    plsc.subcore_barrier()

    plane = cid * (R * N)
    for q in range(nfull):
        o = tid * per_tile_words + q * 1024
        pltpu.sync_copy(cnt_sh.at[pl.ds(o, 1024)], stage.at[pl.ds(0, 1024)])
        pltpu.sync_copy(stage.at[pl.ds(0, 1024)], out_hbm.at[pl.ds(plane + o, 1024)])
    o = tid * per_tile_words + nfull * 1024
    pltpu.sync_copy(cnt_sh.at[pl.ds(o, tail)], stage.at[pl.ds(0, tail)])
    pltpu.sync_copy(stage.at[pl.ds(0, tail)], out_hbm.at[pl.ds(plane + o, tail)])


_counts_call = pl.kernel(
    _sc_counts,
    out_type=jax.ShapeDtypeStruct((2 * R * N,), jnp.float32),
    mesh=_mesh,
    scratch_types=[
        pltpu.VMEM((128,), jnp.int32),
        pltpu.VMEM((128,), jnp.int32),
        pltpu.VMEM((128,), jnp.int32),
        pltpu.VMEM((128,), jnp.float32),
        pltpu.VMEM((128,), jnp.int32),
        pltpu.VMEM((128,), jnp.int32),
        pltpu.VMEM((128,), jnp.int32),
        pltpu.VMEM((128,), jnp.float32),
        pltpu.VMEM((1024,), jnp.float32),
        pltpu.VMEM_SHARED((R * N,), jnp.float32),
        pltpu.SemaphoreType.DMA,
        pltpu.SemaphoreType.DMA,
    ],
)


def _sc_scale(cnt_hbm, dst_hbm, typ_hbm, s_hbm,
              dv, tv, kv, k2v, c0, c1, sv, semld, semg,
              dv1, tv1, kv1, k2v1, c01, c11, sv1, semld1, semg1):
    cid = lax.axis_index("c")
    tid = lax.axis_index("s")
    wid = cid * TPS + tid

    bufs = ((dv, tv, kv, k2v, c0, c1, sv, semld, semg),
            (dv1, tv1, kv1, k2v1, c01, c11, sv1, semld1, semg1))

    def off_of(c):
        cidx = c * 32 + wid
        return jnp.where(cidx < E // 128, cidx * 128, 0)

    def fire(b, c):
        bdv, btv = bufs[b][0], bufs[b][1]
        sem = bufs[b][7]
        off = off_of(c)
        pltpu.async_copy(dst_hbm.at[pl.ds(off, 128)], bdv, sem)
        pltpu.async_copy(typ_hbm.at[pl.ds(off, 128)], btv, sem)

    def wait(b, c):
        bdv, btv = bufs[b][0], bufs[b][1]
        sem = bufs[b][7]
        off = off_of(c)
        pltpu.make_async_copy(dst_hbm.at[pl.ds(off, 128)], bdv, sem).wait()
        pltpu.make_async_copy(typ_hbm.at[pl.ds(off, 128)], btv, sem).wait()

    def process(b, c):
        bdv, btv, bkv, bk2v, bc0, bc1, bsv, _, semgb = bufs[b]
        wait(b, c)
        for g in range(8):
            sl = pl.ds(g * 16, 16)
            key = btv[sl] * N + bdv[sl]
            bkv[sl] = key
            bk2v[sl] = key + R * N
        g0 = pltpu.async_copy(cnt_hbm.at[bkv], bc0, semgb)
        g1 = pltpu.async_copy(cnt_hbm.at[bk2v], bc1, semgb)
        fire(1 - b, c + 1)
        g0.wait()
        g1.wait()
        for g in range(8):
            sl = pl.ds(g * 16, 16)
            tot = bc0[sl] + bc1[sl]
            bsv[sl] = 1.0 / jnp.maximum(tot, 1.0)
        pltpu.sync_copy(bsv, s_hbm.at[pl.ds(off_of(c), 128)])

    fire(0, 0)

    def body(j, _):
        for b in range(2):
            process(b, 2 * j + b)
        return 0

    lax.fori_loop(0, 98, body, 0)
    wait(0, 196)


def _scale_bufs():
    return [
        pltpu.VMEM((128,), jnp.int32),
        pltpu.VMEM((128,), jnp.int32),
        pltpu.VMEM((128,), jnp.int32),
        pltpu.VMEM((128,), jnp.int32),
        pltpu.VMEM((128,), jnp.float32),
        pltpu.VMEM((128,), jnp.float32),
        pltpu.VMEM((128,), jnp.float32),
        pltpu.SemaphoreType.DMA,
        pltpu.SemaphoreType.DMA,
    ]


_scale_call = pl.kernel(
    _sc_scale,
    out_type=jax.ShapeDtypeStruct((E,), jnp.float32),
    mesh=_mesh,
    scratch_types=_scale_bufs() + _scale_bufs(),
)


def _sc_msgpass(y_hbm, src_hbm, dst_hbm, typ_hbm, s_hbm, rpad_hbm, out_hbm,
                srcv0, dv0, tv0, sv0, kv0, dlv0, rows0,
                srcv1, dv1, tv1, sv1, kv1, dlv1, rows1,
                stage, acc, semld0, semld1, semg0, semg1):
    cid = lax.axis_index("c")
    tid = lax.axis_index("s")
    lane = lax.iota(jnp.int32, 16)

    edges_per_tile = E // TPS
    NCH = edges_per_tile // EK
    ebase = tid * edges_per_tile
    dump_row = QN + tid

    bufs = ((srcv0, dv0, tv0, sv0, kv0, dlv0, rows0, semld0, semg0),
            (srcv1, dv1, tv1, sv1, kv1, dlv1, rows1, semld1, semg1))

    def fire_loads(b, off):
        srcv, dv, tv, sv, _, _, _, semld, _ = bufs[b]
        pltpu.async_copy(src_hbm.at[pl.ds(off, EK)], srcv, semld)
        pltpu.async_copy(dst_hbm.at[pl.ds(off, EK)], dv, semld)
        pltpu.async_copy(typ_hbm.at[pl.ds(off, EK)], tv, semld)
        pltpu.async_copy(s_hbm.at[pl.ds(off, EK)], sv, semld)

    def wait_loads(b, off):
        srcv, dv, tv, sv, _, _, _, semld, _ = bufs[b]
        pltpu.make_async_copy(src_hbm.at[pl.ds(off, EK)], srcv, semld).wait()
        pltpu.make_async_copy(dst_hbm.at[pl.ds(off, EK)], dv, semld).wait()
        pltpu.make_async_copy(typ_hbm.at[pl.ds(off, EK)], tv, semld).wait()
        pltpu.make_async_copy(s_hbm.at[pl.ds(off, EK)], sv, semld).wait()

    def finish(b, qbase):
        srcv, dv, tv, sv, kv, dlv, rows, _, semg = bufs[b]
        for g in range(EK // 16):
            sl = pl.ds(g * 16, 16)
            kv[sl] = tv[sl] * N + srcv[sl]
            local = dv[sl] - qbase
            own = (local >= 0) & (local < QN)
            dlv[sl] = jnp.where(own, local, dump_row)
        return pltpu.async_copy(y_hbm.at[kv], rows, semg)

    def drain(b):
        srcv, dv, tv, sv, kv, dlv, rows, _, semg = bufs[b]
        pltpu.make_async_copy(y_hbm.at[kv], rows, semg).wait()
        for g in range(EK // 16):
            svec = sv[pl.ds(g * 16, 16)]
            for l in range(16):
                e = g * 16 + l
                scal = lax.gather(
                    svec, (lane * 0 + l).reshape(16, 1),
                    lax.GatherDimensionNumbers(
                        offset_dims=(), collapsed_slice_dims=(0,),
                        start_index_map=(0,)),
                    (1,), mode=lax.GatherScatterMode.PROMISE_IN_BOUNDS)
                for nb in range(NB):
                    sl = pl.ds(nb * 16, 16)
                    rows[e, sl] = rows[e, sl] * scal
        pltpu.sync_copy(rows, acc.at[dlv], add=True)

    for q in range(2):
        qi = cid * 2 + q
        qbase = qi * QN

        for p in range(NINIT):
            r0 = tid * STRIPE + p * QCHUNK
            pltpu.sync_copy(rpad_hbm.at[qi, pl.ds(r0, QCHUNK), :], stage)
            pltpu.sync_copy(stage, acc.at[pl.ds(r0, QCHUNK), :])

        plsc.subcore_barrier()

        fire_loads(0, ebase)

        def body(j, _):
            for b in range(2):
                jj = 2 * j + b
                off = ebase + jj * EK
                wait_loads(b, off)
                finish(b, qbase)
                fire_loads(1 - b, off + EK)
                drain(b)
            return 0

        lax.fori_loop(0, (NCH - 1) // 2, body, 0)
        off_last = ebase + (NCH - 1) * EK
        wait_loads(0, off_last)
        finish(0, qbase)
        drain(0)

        plsc.subcore_barrier()

        for p in range(NINIT):
            r0 = tid * STRIPE + p * QCHUNK
            pltpu.sync_copy(acc.at[pl.ds(r0, QCHUNK), :], stage)
            pltpu.sync_copy(stage, out_hbm.at[qi, pl.ds(r0, QCHUNK), :])

        if q == 0:
            plsc.subcore_barrier()


def _edge_bufs():
    return [
        pltpu.VMEM((EK,), jnp.int32),
        pltpu.VMEM((EK,), jnp.int32),
        pltpu.VMEM((EK,), jnp.int32),
        pltpu.VMEM((EK,), jnp.float32),
        pltpu.VMEM((EK,), jnp.int32),
        pltpu.VMEM((EK,), jnp.int32),
        pltpu.VMEM((EK, DP), jnp.float32),
    ]


_msgpass_call = pl.kernel(
    _sc_msgpass,
    out_type=jax.ShapeDtypeStruct((4, QROWS, DP), jnp.float32),
    mesh=_mesh,
    scratch_types=(
        _edge_bufs() + _edge_bufs() + [
            pltpu.VMEM((QCHUNK, DP), jnp.float32),
            pltpu.VMEM_SHARED((QROWS, DP), jnp.float32),
            pltpu.SemaphoreType.DMA,
            pltpu.SemaphoreType.DMA,
            pltpu.SemaphoreType.DMA,
            pltpu.SemaphoreType.DMA,
        ]
    ),
)


NT = 2000


def _tc_transform(x_ref, w_ref, root_ref, b_ref, y_ref, r_ref, *, relu_in):
    r = pl.program_id(1)
    x = x_ref[...]
    if relu_in:
        x = jnp.maximum(x, 0.0)
    parts = []
    for b in range(NB):
        xb = x[:, b * BS:(b + 1) * BS]
        parts.append(
            lax.dot_general(xb, w_ref[r, b],
                            (((1,), (0,)), ((), ())),
                            precision=lax.Precision.HIGHEST))
    parts.append(jnp.zeros((NT, DP - H), jnp.float32))
    y_ref[...] = jnp.concatenate(parts, axis=1)

    @pl.when(r == 0)
    def _():
        r_ref[...] = (
            lax.dot_general(x, root_ref[...], (((1,), (0,)), ((), ())),
                            precision=lax.Precision.HIGHEST)
            + b_ref[...]
        )


def _transform(x, w, root, b, relu_in):
    call = pl.pallas_call(
        functools.partial(_tc_transform, relu_in=relu_in),
        grid=(N // NT, R),
        in_specs=[
            pl.BlockSpec((NT, H), lambda i, r: (i, 0)),
            pl.BlockSpec((R, NB, BS, BS), lambda i, r: (0, 0, 0, 0)),
            pl.BlockSpec((H, H), lambda i, r: (0, 0)),
            pl.BlockSpec((1, H), lambda i, r: (0, 0)),
        ],
        out_specs=[
            pl.BlockSpec((NT, DP), lambda i, r: (r * (N // NT) + i, 0)),
            pl.BlockSpec((NT, H), lambda i, r: (i, 0)),
        ],
        out_shape=[
            jax.ShapeDtypeStruct((R * N, DP), jnp.float32),
            jax.ShapeDtypeStruct((N, H), jnp.float32),
        ],
    )
    return call(x, w, root, b.reshape(1, H))


def _pad_root(rterm):
    quarters = rterm.reshape(4, QN, H)
    return jnp.pad(quarters, ((0, 0), (0, QROWS - QN), (0, DP - H)))


def kernel(node_emb, w1, root1, b1, w2, root2, b2, edge_index, edge_type):
    src = edge_index[0].astype(jnp.int32)
    dst = edge_index[1].astype(jnp.int32)
    typ = edge_type.astype(jnp.int32)

    cnt = _counts_call(dst, typ)
    s = _scale_call(cnt, dst, typ)

    y1, rt1 = _transform(node_emb, w1, root1, b1, relu_in=False)
    conv1 = _msgpass_call(y1, src, dst, typ, s, _pad_root(rt1))
    x1 = conv1[:, :QN, :H].reshape(N, H)

    y2, rt2 = _transform(x1, w2, root2, b2, relu_in=True)
    conv2 = _msgpass_call(y2, src, dst, typ, s, _pad_root(rt2))
    return conv2[:, :QN, :H].reshape(N, H)


# speedup vs baseline: 8.8526x; 1.4444x over previous
import functools

import jax
import jax.numpy as jnp
from jax import lax
from jax.experimental import pallas as pl
from jax.experimental.pallas import tpu as pltpu
from jax.experimental.pallas import tpu_sc as plsc

N = 50000
H = 80
DP = 128
R = 8
NB = 5
BS = 16
E = 800000

TPS = 16
QN = N // 4
QROWS = 12544
STRIPE = QROWS // TPS
QCHUNK = STRIPE // 14
NINIT = 14
EK = 80

_mesh = plsc.VectorSubcoreMesh(core_axis_name="c", subcore_axis_name="s")


def _sc_counts(dst_hbm, typ_hbm, out_hbm, dv, tv, kv, vals, dv1, tv1, kv1,
               vals1, stage, cnt_sh, semld, semld1):
    cid = lax.axis_index("c")
    tid = lax.axis_index("s")
    wid = cid * TPS + tid
    per_tile_words = (R * N) // TPS

    z = jnp.zeros((16,), jnp.float32)

    def zbody(i, _):
        stage[pl.ds(i * 16, 16)] = z
        return 0

    lax.fori_loop(0, 64, zbody, 0)
    nfull = per_tile_words // 1024
    for q in range(nfull):
        pltpu.sync_copy(stage.at[pl.ds(0, 1024)],
                        cnt_sh.at[pl.ds(tid * per_tile_words + q * 1024, 1024)])
    tail = per_tile_words - nfull * 1024
    pltpu.sync_copy(stage.at[pl.ds(0, tail)],
                    cnt_sh.at[pl.ds(tid * per_tile_words + nfull * 1024, tail)])

    plsc.subcore_barrier()

    bufs = ((dv, tv, kv, vals, semld), (dv1, tv1, kv1, vals1, semld1))

    def off_of(c):
        cidx = c * 32 + wid
        return jnp.where(cidx < E // 128, cidx * 128, 0), cidx < E // 128

    def fire(b, c):
        bdv, btv, _, _, sem = bufs[b]
        off, _ = off_of(c)
        pltpu.async_copy(dst_hbm.at[pl.ds(off, 128)], bdv, sem)
        pltpu.async_copy(typ_hbm.at[pl.ds(off, 128)], btv, sem)

    def wait(b, c):
        bdv, btv, _, _, sem = bufs[b]
        off, _ = off_of(c)
        pltpu.make_async_copy(dst_hbm.at[pl.ds(off, 128)], bdv, sem).wait()
        pltpu.make_async_copy(typ_hbm.at[pl.ds(off, 128)], btv, sem).wait()

    def process(b, c):
        bdv, btv, bkv, bvals, _ = bufs[b]
        _, valid = off_of(c)
        vf = jnp.where(valid, 1.0, 0.0)
        for g in range(8):
            sl = pl.ds(g * 16, 16)
            bkv[sl] = btv[sl] * N + bdv[sl]
            bvals[sl] = jnp.zeros((16,), jnp.float32) + vf
        pltpu.sync_copy(bvals, cnt_sh.at[bkv], add=True)

    fire(0, 0)

    def body(j, _):
        for b in range(2):
            c = 2 * j + b
            wait(b, c)
            fire(1 - b, c + 1)
            process(b, c)
        return 0

    lax.fori_loop(0, 98, body, 0)
    wait(0, 196)

    plsc.subcore_barrier()

    plane = cid * (R * N)
    for q in range(nfull):
        o = tid * per_tile_words + q * 1024
        pltpu.sync_copy(cnt_sh.at[pl.ds(o, 1024)], stage.at[pl.ds(0, 1024)])
        pltpu.sync_copy(stage.at[pl.ds(0, 1024)], out_hbm.at[pl.ds(plane + o, 1024)])
    o = tid * per_tile_words + nfull * 1024
    pltpu.sync_copy(cnt_sh.at[pl.ds(o, tail)], stage.at[pl.ds(0, tail)])
    pltpu.sync_copy(stage.at[pl.ds(0, tail)], out_hbm.at[pl.ds(plane + o, tail)])


_counts_call = pl.kernel(
    _sc_counts,
    out_type=jax.ShapeDtypeStruct((2 * R * N,), jnp.float32),
    mesh=_mesh,
    scratch_types=[
        pltpu.VMEM((128,), jnp.int32),
        pltpu.VMEM((128,), jnp.int32),
        pltpu.VMEM((128,), jnp.int32),
        pltpu.VMEM((128,), jnp.float32),
        pltpu.VMEM((128,), jnp.int32),
        pltpu.VMEM((128,), jnp.int32),
        pltpu.VMEM((128,), jnp.int32),
        pltpu.VMEM((128,), jnp.float32),
        pltpu.VMEM((1024,), jnp.float32),
        pltpu.VMEM_SHARED((R * N,), jnp.float32),
        pltpu.SemaphoreType.DMA,
        pltpu.SemaphoreType.DMA,
    ],
)


def _sc_scale(cnt_hbm, dst_hbm, typ_hbm, s_hbm,
              dv, tv, kv, k2v, c0, c1, sv, semld, semg,
              dv1, tv1, kv1, k2v1, c01, c11, sv1, semld1, semg1):
    cid = lax.axis_index("c")
    tid = lax.axis_index("s")
    wid = cid * TPS + tid

    bufs = ((dv, tv, kv, k2v, c0, c1, sv, semld, semg),
            (dv1, tv1, kv1, k2v1, c01, c11, sv1, semld1, semg1))

    def off_of(c):
        cidx = c * 32 + wid
        return jnp.where(cidx < E // 128, cidx * 128, 0)

    def fire(b, c):
        bdv, btv = bufs[b][0], bufs[b][1]
        sem = bufs[b][7]
        off = off_of(c)
        pltpu.async_copy(dst_hbm.at[pl.ds(off, 128)], bdv, sem)
        pltpu.async_copy(typ_hbm.at[pl.ds(off, 128)], btv, sem)

    def wait(b, c):
        bdv, btv = bufs[b][0], bufs[b][1]
        sem = bufs[b][7]
        off = off_of(c)
        pltpu.make_async_copy(dst_hbm.at[pl.ds(off, 128)], bdv, sem).wait()
        pltpu.make_async_copy(typ_hbm.at[pl.ds(off, 128)], btv, sem).wait()

    def process(b, c):
        bdv, btv, bkv, bk2v, bc0, bc1, bsv, _, semgb = bufs[b]
        wait(b, c)
        for g in range(8):
            sl = pl.ds(g * 16, 16)
            key = btv[sl] * N + bdv[sl]
            bkv[sl] = key
            bk2v[sl] = key + R * N
        g0 = pltpu.async_copy(cnt_hbm.at[bkv], bc0, semgb)
        g1 = pltpu.async_copy(cnt_hbm.at[bk2v], bc1, semgb)
        fire(1 - b, c + 1)
        g0.wait()
        g1.wait()
        for g in range(8):
            sl = pl.ds(g * 16, 16)
            tot = bc0[sl] + bc1[sl]
            bsv[sl] = 1.0 / jnp.maximum(tot, 1.0)
        pltpu.sync_copy(bsv, s_hbm.at[pl.ds(off_of(c), 128)])

    fire(0, 0)

    def body(j, _):
        for b in range(2):
            process(b, 2 * j + b)
        return 0

    lax.fori_loop(0, 98, body, 0)
    wait(0, 196)


def _scale_bufs():
    return [
        pltpu.VMEM((128,), jnp.int32),
        pltpu.VMEM((128,), jnp.int32),
        pltpu.VMEM((128,), jnp.int32),
        pltpu.VMEM((128,), jnp.int32),
        pltpu.VMEM((128,), jnp.float32),
        pltpu.VMEM((128,), jnp.float32),
        pltpu.VMEM((128,), jnp.float32),
        pltpu.SemaphoreType.DMA,
        pltpu.SemaphoreType.DMA,
    ]


_scale_call = pl.kernel(
    _sc_scale,
    out_type=jax.ShapeDtypeStruct((E,), jnp.float32),
    mesh=_mesh,
    scratch_types=_scale_bufs() + _scale_bufs(),
)


def _sc_msgpass(y_hbm, src_hbm, dst_hbm, typ_hbm, s_hbm, rpad_hbm, out_hbm,
                srcv0, dv0, tv0, sv0, kv0, dlv0, rows0, svd0,
                srcv1, dv1, tv1, sv1, kv1, dlv1, rows1, svd1,
                stage, acc, semld0, semld1, semg0, semg1):
    cid = lax.axis_index("c")
    tid = lax.axis_index("s")
    lane = lax.iota(jnp.int32, 16)

    edges_per_tile = E // TPS
    NCH = edges_per_tile // EK
    ebase = tid * edges_per_tile
    dump_row = QN + tid

    bufs = ((srcv0, dv0, tv0, sv0, kv0, dlv0, rows0, svd0, semld0, semg0),
            (srcv1, dv1, tv1, sv1, kv1, dlv1, rows1, svd1, semld1, semg1))

    def fire_loads(b, off):
        srcv, dv, tv, sv, _, _, _, _, semld, _ = bufs[b]
        pltpu.async_copy(src_hbm.at[pl.ds(off, EK)], srcv, semld)
        pltpu.async_copy(dst_hbm.at[pl.ds(off, EK)], dv, semld)
        pltpu.async_copy(typ_hbm.at[pl.ds(off, EK)], tv, semld)
        pltpu.async_copy(s_hbm.at[pl.ds(off, EK)], sv, semld)

    def wait_loads(b, off):
        srcv, dv, tv, sv, _, _, _, _, semld, _ = bufs[b]
        pltpu.make_async_copy(src_hbm.at[pl.ds(off, EK)], srcv, semld).wait()
        pltpu.make_async_copy(dst_hbm.at[pl.ds(off, EK)], dv, semld).wait()
        pltpu.make_async_copy(typ_hbm.at[pl.ds(off, EK)], tv, semld).wait()
        pltpu.make_async_copy(s_hbm.at[pl.ds(off, EK)], sv, semld).wait()

    def finish(b, qbase):
        srcv, dv, tv, sv, kv, dlv, rows, svd, _, semg = bufs[b]
        for g in range(EK // 16):
            sl = pl.ds(g * 16, 16)
            kv[sl] = tv[sl] * N + srcv[sl]
            local = dv[sl] - qbase
            own = (local >= 0) & (local < QN)
            dlv[sl] = jnp.where(own, local, dump_row)
            svd[sl] = sv[sl]
        return pltpu.async_copy(y_hbm.at[kv], rows, semg)

    def drain(b):
        srcv, dv, tv, sv, kv, dlv, rows, svd, _, semg = bufs[b]
        pltpu.make_async_copy(y_hbm.at[kv], rows, semg).wait()
        for g in range(EK // 16):
            svec = svd[pl.ds(g * 16, 16)]
            for l in range(16):
                e = g * 16 + l
                scal = lax.gather(
                    svec, (lane * 0 + l).reshape(16, 1),
                    lax.GatherDimensionNumbers(
                        offset_dims=(), collapsed_slice_dims=(0,),
                        start_index_map=(0,)),
                    (1,), mode=lax.GatherScatterMode.PROMISE_IN_BOUNDS)
                for nb in range(NB):
                    sl = pl.ds(nb * 16, 16)
                    rows[e, sl] = rows[e, sl] * scal
        pltpu.sync_copy(rows, acc.at[dlv], add=True)

    for q in range(2):
        qi = cid * 2 + q
        qbase = qi * QN

        for p in range(NINIT):
            r0 = tid * STRIPE + p * QCHUNK
            pltpu.sync_copy(rpad_hbm.at[qi, pl.ds(r0, QCHUNK), :], stage)
            pltpu.sync_copy(stage, acc.at[pl.ds(r0, QCHUNK), :])

        plsc.subcore_barrier()

        fire_loads(0, ebase)
        wait_loads(0, ebase)
        finish(0, qbase)
        fire_loads(1, ebase + EK)

        def body(j, _):
            o1 = ebase + (2 * j + 1) * EK
            o2 = ebase + jnp.minimum(2 * j + 2, NCH - 1) * EK
            o3 = ebase + jnp.minimum(2 * j + 3, NCH - 1) * EK
            wait_loads(1, o1)
            finish(1, qbase)
            fire_loads(0, o2)
            drain(0)
            wait_loads(0, o2)
            finish(0, qbase)
            fire_loads(1, o3)
            drain(1)
            return 0

        lax.fori_loop(0, NCH // 2, body, 0)
        drain(0)
        wait_loads(1, ebase + (NCH - 1) * EK)

        plsc.subcore_barrier()

        for p in range(NINIT):
            r0 = tid * STRIPE + p * QCHUNK
            pltpu.sync_copy(acc.at[pl.ds(r0, QCHUNK), :], stage)
            pltpu.sync_copy(stage, out_hbm.at[qi, pl.ds(r0, QCHUNK), :])

        if q == 0:
            plsc.subcore_barrier()


def _edge_bufs():
    return [
        pltpu.VMEM((EK,), jnp.int32),
        pltpu.VMEM((EK,), jnp.int32),
        pltpu.VMEM((EK,), jnp.int32),
        pltpu.VMEM((EK,), jnp.float32),
        pltpu.VMEM((EK,), jnp.int32),
        pltpu.VMEM((EK,), jnp.int32),
        pltpu.VMEM((EK, DP), jnp.float32),
        pltpu.VMEM((EK,), jnp.float32),
    ]


_msgpass_call = pl.kernel(
    _sc_msgpass,
    out_type=jax.ShapeDtypeStruct((4, QROWS, DP), jnp.float32),
    mesh=_mesh,
    scratch_types=(
        _edge_bufs() + _edge_bufs() + [
            pltpu.VMEM((QCHUNK, DP), jnp.float32),
            pltpu.VMEM_SHARED((QROWS, DP), jnp.float32),
            pltpu.SemaphoreType.DMA,
            pltpu.SemaphoreType.DMA,
            pltpu.SemaphoreType.DMA,
            pltpu.SemaphoreType.DMA,
        ]
    ),
)


NT = 2000


def _tc_transform(x_ref, w_ref, root_ref, b_ref, y_ref, r_ref, *, relu_in):
    r = pl.program_id(1)
    x = x_ref[...]
    if relu_in:
        x = jnp.maximum(x, 0.0)
    parts = []
    for b in range(NB):
        xb = x[:, b * BS:(b + 1) * BS]
        parts.append(
            lax.dot_general(xb, w_ref[r, b],
                            (((1,), (0,)), ((), ())),
                            precision=lax.Precision.HIGHEST))
    parts.append(jnp.zeros((NT, DP - H), jnp.float32))
    y_ref[...] = jnp.concatenate(parts, axis=1)

    @pl.when(r == 0)
    def _():
        r_ref[...] = (
            lax.dot_general(x, root_ref[...], (((1,), (0,)), ((), ())),
                            precision=lax.Precision.HIGHEST)
            + b_ref[...]
        )


def _transform(x, w, root, b, relu_in):
    call = pl.pallas_call(
        functools.partial(_tc_transform, relu_in=relu_in),
        grid=(N // NT, R),
        in_specs=[
            pl.BlockSpec((NT, H), lambda i, r: (i, 0)),
            pl.BlockSpec((R, NB, BS, BS), lambda i, r: (0, 0, 0, 0)),
            pl.BlockSpec((H, H), lambda i, r: (0, 0)),
            pl.BlockSpec((1, H), lambda i, r: (0, 0)),
        ],
        out_specs=[
            pl.BlockSpec((NT, DP), lambda i, r: (r * (N // NT) + i, 0)),
            pl.BlockSpec((NT, H), lambda i, r: (i, 0)),
        ],
        out_shape=[
            jax.ShapeDtypeStruct((R * N, DP), jnp.float32),
            jax.ShapeDtypeStruct((N, H), jnp.float32),
        ],
    )
    return call(x, w, root, b.reshape(1, H))


def _pad_root(rterm):
    quarters = rterm.reshape(4, QN, H)
    return jnp.pad(quarters, ((0, 0), (0, QROWS - QN), (0, DP - H)))


def kernel(node_emb, w1, root1, b1, w2, root2, b2, edge_index, edge_type):
    src = edge_index[0].astype(jnp.int32)
    dst = edge_index[1].astype(jnp.int32)
    typ = edge_type.astype(jnp.int32)

    cnt = _counts_call(dst, typ)
    s = _scale_call(cnt, dst, typ)

    y1, rt1 = _transform(node_emb, w1, root1, b1, relu_in=False)
    conv1 = _msgpass_call(y1, src, dst, typ, s, _pad_root(rt1))
    x1 = conv1[:, :QN, :H].reshape(N, H)

    y2, rt2 = _transform(x1, w2, root2, b2, relu_in=True)
    conv2 = _msgpass_call(y2, src, dst, typ, s, _pad_root(rt2))
    return conv2[:, :QN, :H].reshape(N, H)
